# baseline (speedup 1.0000x reference)
import functools

import jax
import jax.numpy as jnp
from jax import lax
from jax.experimental import pallas as pl
from jax.experimental.pallas import tpu as pltpu
from jax.experimental.pallas import tpu_sc as plsc

N = 10000
NPAD = 10240
NE = 320000
K = 8
NC = 2
NS = 16
NW = NC * NS
ET = NE // 128
ETP = 2560
NE2 = ETP * 128
EPT = NE2 // NW
TCH = 16
CHUNK = TCH * 128
NCHUNKS = (ETP // NW) // TCH
CHB = 1024
NCHB = EPT // CHB
SLICE = NPAD // NS
F = 16

_mesh = plsc.VectorSubcoreMesh(core_axis_name="c", subcore_axis_name="s")
_sc_params = pltpu.CompilerParams(
    needs_layout_passes=False, use_tc_tiling_on_sc=False)


def _tc_dense(x_pad, rv, W):
    din = x_pad.shape[1]

    def kfn(x_ref, rv_ref, w_ref, p_ref, hw_ref):
        xb = x_ref[...]
        p_ref[...] = jnp.dot(xb, rv_ref[...], preferred_element_type=jnp.float32)
        hw_ref[...] = jnp.dot(xb, w_ref[...], preferred_element_type=jnp.float32)

    return pl.pallas_call(
        kfn,
        grid=(NPAD // 2048,),
        in_specs=[
            pl.BlockSpec((2048, din), lambda i: (i, 0)),
            pl.BlockSpec((din, 1), lambda i: (0, 0)),
            pl.BlockSpec((din, F), lambda i: (0, 0)),
        ],
        out_specs=[
            pl.BlockSpec((2048, 1), lambda i: (i, 0)),
            pl.BlockSpec((2048, F), lambda i: (i, 0)),
        ],
        out_shape=[
            jax.ShapeDtypeStruct((NPAD, 1), jnp.float32),
            jax.ShapeDtypeStruct((NPAD, F), jnp.float32),
        ],
    )(x_pad, rv.reshape(din, 1), W)


@functools.partial(
    pl.kernel,
    out_type=[
        jax.ShapeDtypeStruct((NE2,), jnp.int32),
        jax.ShapeDtypeStruct((NE2,), jnp.int32),
        jax.ShapeDtypeStruct((NC * NPAD,), jnp.float32),
    ],
    mesh=_mesh,
    compiler_params=_sc_params,
    scratch_types=[
        pltpu.VMEM((NPAD,), jnp.float32),
        pltpu.VMEM((2, TCH, K, 128), jnp.int32),
        pltpu.VMEM((2, CHUNK), jnp.int32),
        pltpu.VMEM((2, CHUNK), jnp.int32),
        pltpu.VMEM((CHUNK,), jnp.float32),
        pltpu.VMEM((SLICE,), jnp.float32),
        pltpu.VMEM_SHARED((NPAD,), jnp.float32),
        pltpu.SemaphoreType.DMA,
        pltpu.SemaphoreType.DMA,
        pltpu.SemaphoreType.DMA,
        pltpu.SemaphoreType.DMA,
    ],
)
def _sc_edges(proj_hbm, edges_hbm, se_hbm, ie_hbm, deg_hbm,
              proj_v, ebuf, sebuf, iebuf, valbuf, stage, degsh,
              lsem0, lsem1, ssem0, ssem1):
    cid = lax.axis_index("c")
    sid = lax.axis_index("s")
    wid = cid * NS + sid
    ebase = wid * EPT
    tbase = wid * (ETP // NW)
    lsem = (lsem0, lsem1)
    ssem = (ssem0, ssem1)

    pltpu.sync_copy(proj_hbm, proj_v)

    def init_val(i, _):
        valbuf[pl.ds(i * 16, 16)] = jnp.full((16,), 0.125, jnp.float32)
        return 0

    lax.fori_loop(0, CHUNK // 16, init_val, 0)

    def init_zero(i, _):
        stage[pl.ds(i * 16, 16)] = jnp.zeros((16,), jnp.float32)
        return 0

    lax.fori_loop(0, SLICE // 16, init_zero, 0)
    pltpu.sync_copy(stage, degsh.at[pl.ds(sid * SLICE, SLICE)])
    plsc.subcore_barrier()

    def load_edges(c, b):
        return pltpu.async_copy(
            edges_hbm.at[pl.ds(tbase + c * TCH, TCH)], ebuf.at[b], lsem[b])

    pend_load = {0: load_edges(0, 0), 1: load_edges(1, 1)}
    pend_store = {}

    for c in range(NCHUNKS):
        b = c % 2
        for d in pend_store.pop(c - 2, ()):
            d.wait()
        pend_load.pop(c).wait()
        eb = ebuf.at[b]
        seb = sebuf.at[b]
        ieb = iebuf.at[b]

        @plsc.parallel_loop(0, CHUNK // 16, unroll=4)
        def _(g):
            t = g // 8
            l0 = (g % 8) * 16
            n_cur = eb[t, 0, pl.ds(l0, 16)]
            p_cur = plsc.load_gather(proj_v, [n_cur])
            nmax = n_cur
            pmax = p_cur
            nmin = n_cur
            pmin = p_cur
            for j in range(1, K):
                nj = eb[t, j, pl.ds(l0, 16)]
                pj = plsc.load_gather(proj_v, [nj])
                gt = pj > pmax
                nmax = jnp.where(gt, nj, nmax)
                pmax = jnp.where(gt, pj, pmax)
                ltm = pj < pmin
                nmin = jnp.where(ltm, nj, nmin)
                pmin = jnp.where(ltm, pj, pmin)
            seb[pl.ds(g * 16, 16)] = nmax
            ieb[pl.ds(g * 16, 16)] = nmin

        pltpu.sync_copy(valbuf, degsh.at[seb], add=True)
        pltpu.sync_copy(valbuf, degsh.at[ieb], add=True)
        pend_store[c] = (
            pltpu.async_copy(
                seb, se_hbm.at[pl.ds(ebase + c * CHUNK, CHUNK)], ssem[b]),
            pltpu.async_copy(
                ieb, ie_hbm.at[pl.ds(ebase + c * CHUNK, CHUNK)], ssem[b]),
        )
        if c + 2 < NCHUNKS:
            pend_load[c + 2] = load_edges(c + 2, b)

    for c in (NCHUNKS - 2, NCHUNKS - 1):
        for d in pend_store.pop(c, ()):
            d.wait()

    plsc.subcore_barrier()
    pltpu.sync_copy(degsh.at[pl.ds(sid * SLICE, SLICE)], stage)
    pltpu.sync_copy(stage, deg_hbm.at[pl.ds(cid * NPAD + sid * SLICE, SLICE)])


def _tc_norm(degp, hw):
    def kfn(d_ref, hw_ref, dinv_ref, gs_ref):
        deg = d_ref[0] + d_ref[1] + 1.0
        dinv = lax.rsqrt(deg)
        dinv_ref[...] = dinv
        gs_ref[...] = dinv * hw_ref[...] * 0.125

    return pl.pallas_call(
        kfn,
        grid=(NPAD // 2048,),
        in_specs=[
            pl.BlockSpec((2, 2048, 1), lambda i: (0, i, 0)),
            pl.BlockSpec((2048, F), lambda i: (i, 0)),
        ],
        out_specs=[
            pl.BlockSpec((2048, 1), lambda i: (i, 0)),
            pl.BlockSpec((2048, F), lambda i: (i, 0)),
        ],
        out_shape=[
            jax.ShapeDtypeStruct((NPAD, 1), jnp.float32),
            jax.ShapeDtypeStruct((NPAD, F), jnp.float32),
        ],
    )(degp, hw)


@functools.partial(
    pl.kernel,
    out_type=jax.ShapeDtypeStruct((NC, NS, SLICE, F), jnp.float32),
    mesh=_mesh,
    compiler_params=_sc_params,
    scratch_types=[
        pltpu.VMEM((NCHB, CHB), jnp.int32),
        pltpu.VMEM((NCHB, CHB), jnp.int32),
        pltpu.VMEM((2, CHB, F), jnp.float32),
        pltpu.VMEM((2, CHB, F), jnp.float32),
        pltpu.VMEM((SLICE, F), jnp.float32),
        pltpu.VMEM_SHARED((NPAD, F), jnp.float32),
        pltpu.SemaphoreType.DMA,
        pltpu.SemaphoreType.DMA,
        pltpu.SemaphoreType.DMA,
        pltpu.SemaphoreType.DMA,
        pltpu.SemaphoreType.DMA,
        pltpu.SemaphoreType.DMA,
        pltpu.SemaphoreType.DMA,
    ],
)
def _sc_msgs(se_hbm, ie_hbm, gs_hbm, acc_hbm,
             sebuf, iebuf, rbufa, rbufb, stage, accsh,
             isem, gsem0, gsem1, hsem0, hsem1, csem0, csem1):
    cid = lax.axis_index("c")
    sid = lax.axis_index("s")
    ebase = (cid * NS + sid) * EPT
    rbase = sid * SLICE
    gsem = (gsem0, gsem1)
    hsem = (hsem0, hsem1)
    csem = (csem0, csem1)

    idx_loads = []
    for c in range(NCHB):
        idx_loads.append(pltpu.async_copy(
            se_hbm.at[pl.ds(ebase + c * CHB, CHB)], sebuf.at[c], isem))
        idx_loads.append(pltpu.async_copy(
            ie_hbm.at[pl.ds(ebase + c * CHB, CHB)], iebuf.at[c], isem))

    def init_zero(i, _):
        stage[i, :] = jnp.zeros((F,), jnp.float32)
        return 0

    lax.fori_loop(0, SLICE, init_zero, 0)
    pltpu.sync_copy(stage, accsh.at[pl.ds(rbase, SLICE)])
    for d in idx_loads:
        d.wait()
    plsc.subcore_barrier()

    pend_sca = {}
    for c in range(NCHB):
        b = c % 2
        for d in pend_sca.pop(c - 2, ()):
            d.wait()
        seb = sebuf.at[c]
        ieb = iebuf.at[c]
        ra = rbufa.at[b]
        rb = rbufb.at[b]
        ga = pltpu.async_copy(gs_hbm.at[ieb], ra, gsem[b])
        gb = pltpu.async_copy(gs_hbm.at[seb], rb, hsem[b])
        ga.wait()
        s1 = pltpu.async_copy(ra, accsh.at[seb], csem[b], add=True)
        gb.wait()
        s2 = pltpu.async_copy(rb, accsh.at[ieb], csem[b], add=True)
        pend_sca[c] = (s1, s2)

    for c in (NCHB - 2, NCHB - 1):
        for d in pend_sca.pop(c, ()):
            d.wait()

    plsc.subcore_barrier()
    pltpu.sync_copy(accsh.at[pl.ds(rbase, SLICE)], stage)
    pltpu.sync_copy(stage, acc_hbm.at[cid, sid])


def _tc_combine(accp, dinv, gs, b):
    def kfn(a_ref, dinv_ref, gs_ref, b_ref, o_ref):
        acc = a_ref[0] + a_ref[1]
        o_ref[...] = jnp.maximum(
            dinv_ref[...] * (acc + 8.0 * gs_ref[...]) + b_ref[...], 0.0)

    return pl.pallas_call(
        kfn,
        grid=(NPAD // 2048,),
        in_specs=[
            pl.BlockSpec((2, 2048, F), lambda i: (0, i, 0)),
            pl.BlockSpec((2048, 1), lambda i: (i, 0)),
            pl.BlockSpec((2048, F), lambda i: (i, 0)),
            pl.BlockSpec((1, F), lambda i: (0, 0)),
        ],
        out_specs=pl.BlockSpec((2048, F), lambda i: (i, 0)),
        out_shape=jax.ShapeDtypeStruct((NPAD, F), jnp.float32),
    )(accp, dinv, gs, b)


def _layer(H_pad, edges3d, rv, W, b):
    proj, hw = _tc_dense(H_pad, rv, W)
    se, ie, degf = _sc_edges(proj.reshape(NPAD), edges3d)
    dinv, gs = _tc_norm(degf.reshape(NC, NPAD, 1), hw)
    accp = _sc_msgs(se, ie, gs)
    return _tc_combine(accp.reshape(NC, NPAD, F), dinv, gs, b.reshape(1, F))


def kernel(x, hyperedges, rv1, rv2, W1, b1, W2, b2):
    x_pad = jnp.concatenate(
        [x, jnp.zeros((NPAD - N, x.shape[1]), x.dtype)], axis=0)
    edges3d = jnp.transpose(hyperedges.reshape(ET, 128, K), (0, 2, 1))
    edges3d = jnp.pad(edges3d, ((0, ETP - ET), (0, 0), (0, 0)),
                      constant_values=NPAD - 1)
    H = _layer(x_pad, edges3d, rv1, W1, b1)
    H = _layer(H, edges3d, rv2, W2, b2)
    return H[:N]
